# baseline (speedup 1.0000x reference)
import jax
import jax.numpy as jnp
from jax.experimental import pallas as pl
from jax.experimental.pallas import tpu as pltpu

_N = 2048
_H = 12
_R = 32
_C = 128


def _sort_kernel(x_ref, c_ref, out_ref, key_scratch, xrow, sem):
    cp = pltpu.make_async_copy(x_ref.at[:, :, 0, :], xrow, sem)
    cp.start()
    cp.wait()
    for b in range(2):
        w = xrow[b, 0:1, :]
        c = c_ref[b, 0:1, :]
        for h in range(1, _H):
            w = w + xrow[b, h:h + 1, :]
            c = c + c_ref[b, h:h + 1, :]
        s = (c / float(_H)) * (w / float(_H))
        s = s + 0.0
        ib = jax.lax.bitcast_convert_type(s, jnp.int32)
        key_row = ib ^ ((ib >> 31) & jnp.int32(0x7FFFFFFF))
        for i in range(16):
            key_scratch[b * 16 + i:b * 16 + i + 1, :] = \
                key_row[:, _C * i:_C * (i + 1)]
    key = key_scratch[...]

    rows = jax.lax.broadcasted_iota(jnp.int32, (_R, _C), 0)
    cols = jax.lax.broadcasted_iota(jnp.int32, (_R, _C), 1)
    p = cols * 16 + (rows & 15)
    idx = (rows & 15) * _C + cols

    blk = 2
    while blk <= _N:
        asc = (p & blk) == 0
        d = blk // 2
        while d >= 1:
            lower = (p & d) == 0
            sel = lower == asc
            if d >= 16:
                e = d // 16
                pk = jnp.where(lower, jnp.roll(key, -e, axis=1),
                               jnp.roll(key, e, axis=1))
                pi = jnp.where(lower, jnp.roll(idx, -e, axis=1),
                               jnp.roll(idx, e, axis=1))
            else:
                pk = jnp.where(lower, jnp.roll(key, -d, axis=0),
                               jnp.roll(key, d, axis=0))
                pi = jnp.where(lower, jnp.roll(idx, -d, axis=0),
                               jnp.roll(idx, d, axis=0))
            v_first = (key > pk) | ((key == pk) & (idx < pi))
            keep_v = v_first == sel
            key = jnp.where(keep_v, key, pk)
            idx = jnp.where(keep_v, idx, pi)
            d //= 2
        blk *= 2

    t = jnp.swapaxes(idx, 0, 1)
    out_ref[0:1] = t[:, 0:16].reshape(1, _C, 16)
    out_ref[1:2] = t[:, 16:32].reshape(1, _C, 16)


@jax.jit
def kernel(x, contributions):
    b = x.shape[0]
    return pl.pallas_call(
        _sort_kernel,
        grid=(1,),
        in_specs=[
            pl.BlockSpec(memory_space=pltpu.MemorySpace.HBM),
            pl.BlockSpec((b, _H, _N), lambda i: (0, 0, 0)),
        ],
        out_specs=pl.BlockSpec((b, _C, 16), lambda i: (0, 0, 0)),
        out_shape=jax.ShapeDtypeStruct((b, _C, 16), jnp.int32),
        scratch_shapes=[pltpu.VMEM((_R, _C), jnp.int32),
                        pltpu.VMEM((2, _H, _N), jnp.float32),
                        pltpu.SemaphoreType.DMA],
    )(x, contributions).reshape(b, _N)
